# baseline (speedup 1.0000x reference)
import functools

import jax
import jax.numpy as jnp
from jax import lax
from jax.experimental import pallas as pl
from jax.experimental.pallas import tpu as pltpu
from jax.experimental.pallas import tpu_sc as plsc

N_DAYS = 16384
N_TIMES = 390
F = 8
WINDOW = 60
ROW = 80
ROWS_PER_DAY = (N_TIMES * F) // ROW
ROWS_PER_B = (WINDOW * F) // ROW
L = 16
NC, NS = 2, 16
NW = NC * NS

CHUNK_B = 128
R_CHUNK = CHUNK_B * ROWS_PER_B
N_SUB = R_CHUNK // 128


def _make_gather(batch):
    b_per_w = batch // NW
    n_chunks = b_per_w // CHUNK_B
    mesh = plsc.VectorSubcoreMesh(
        core_axis_name="c", subcore_axis_name="s",
        num_cores=NC, num_subcores=NS)

    @functools.partial(
        pl.kernel,
        mesh=mesh,
        out_type=jax.ShapeDtypeStruct((batch * ROWS_PER_B, ROW), jnp.float32),
        scratch_types=[
            pltpu.VMEM((b_per_w,), jnp.int32),
            pltpu.VMEM((N_SUB, 128), jnp.int32),
            pltpu.VMEM((R_CHUNK, ROW), jnp.float32),
            pltpu.VMEM((L,), jnp.int32),
            pltpu.SemaphoreType.DMA,
        ],
        compiler_params=pltpu.CompilerParams(use_tc_tiling_on_sc=False),
    )
    def k(table, didx, row0, out, didx_v, idx_v, rows_v, r0_v, sem):
        wid = lax.axis_index("s") * NC + lax.axis_index("c")
        base_b = wid * b_per_w
        pltpu.sync_copy(didx.at[pl.ds(base_b, b_per_w)], didx_v)
        pltpu.sync_copy(row0, r0_v)
        r0vec = r0_v[...]
        lanes = lax.iota(jnp.int32, L)
        six = jnp.full((L,), ROWS_PER_B, jnp.int32)
        qpat, rmpat = [], []
        for t in range(ROWS_PER_B):
            kv = lanes + (t * L)
            q = lax.div(kv, six)
            qpat.append(q)
            rmpat.append(kv - q * six)
        for c in range(n_chunks):
            for g in range(CHUNK_B // L):
                dvec = didx_v[pl.ds(c * CHUNK_B + g * L, L)]
                base = dvec * ROWS_PER_DAY + r0vec
                for t in range(ROWS_PER_B):
                    val = base.at[qpat[t]].get(
                        mode="promise_in_bounds") + rmpat[t]
                    k0 = g * (L * ROWS_PER_B) + t * L
                    idx_v[k0 // 128, pl.ds(k0 % 128, L)] = val
            copies = [
                pltpu.async_copy(table.at[idx_v.at[s]],
                                 rows_v.at[pl.ds(s * 128, 128)], sem)
                for s in range(N_SUB)
            ]
            for cp in copies:
                cp.wait()
            pltpu.sync_copy(
                rows_v,
                out.at[pl.ds(base_b * ROWS_PER_B + c * R_CHUNK, R_CHUNK)])

    return k


def kernel(price_data, date_idx, time_idx):
    batch = date_idx.shape[0]
    table = price_data.reshape(N_DAYS * ROWS_PER_DAY, ROW)
    row0 = (time_idx * F - WINDOW * F) // ROW
    row0_arr = jnp.full((L,), row0, dtype=jnp.int32)
    didx = date_idx.astype(jnp.int32)
    out = _make_gather(batch)(table, didx, row0_arr)
    return out.reshape(batch, WINDOW, F)


# speedup vs baseline: 18.6630x; 18.6630x over previous
import functools

import jax
import jax.numpy as jnp
from jax import lax
from jax.experimental import pallas as pl
from jax.experimental.pallas import tpu as pltpu
from jax.experimental.pallas import tpu_sc as plsc

N_DAYS = 16384
N_TIMES = 390
F = 8
WINDOW = 60
L = 16
NC, NS = 2, 16
NW = NC * NS
N_ROWS = WINDOW * F
ROWS_PER_W = N_ROWS // NW


def _make_gather(batch):
    n_vec = batch // L
    mesh = plsc.VectorSubcoreMesh(
        core_axis_name="c", subcore_axis_name="s",
        num_cores=NC, num_subcores=NS)

    @functools.partial(
        pl.kernel,
        mesh=mesh,
        out_type=jax.ShapeDtypeStruct((N_ROWS, batch), jnp.float32),
        scratch_types=[
            pltpu.VMEM((batch,), jnp.int32),
            pltpu.VMEM((batch,), jnp.float32),
            pltpu.VMEM((batch,), jnp.float32),
            pltpu.VMEM((batch,), jnp.float32),
            pltpu.VMEM((L,), jnp.int32),
            pltpu.SemaphoreType.DMA,
            pltpu.SemaphoreType.DMA,
        ],
        compiler_params=pltpu.CompilerParams(needs_layout_passes=False),
    )
    def k(table, didx, row0, out, didx_v, slab_a, slab_b, orow_v, r0_v,
          ld_sem, st_sem):
        wid = lax.axis_index("s") * NC + lax.axis_index("c")
        s0 = wid * ROWS_PER_W
        pltpu.sync_copy(didx.at[pl.ds(0, batch)], didx_v)
        pltpu.sync_copy(row0, r0_v)
        r0s = jnp.min(r0_v[...])
        slabs = [slab_a, slab_b]
        copies = [None, None]
        copies[0] = pltpu.async_copy(
            table.at[r0s + s0], slabs[0], ld_sem)
        st = None
        for i in range(ROWS_PER_W):
            cur = i % 2
            copies[cur].wait()
            if i + 1 < ROWS_PER_W:
                copies[1 - cur] = pltpu.async_copy(
                    table.at[r0s + s0 + (i + 1)], slabs[1 - cur], ld_sem)
            slab = slabs[cur]

            def body(v, _):
                idx = didx_v[pl.ds(v * L, L)]
                orow_v[pl.ds(v * L, L)] = plsc.load_gather(slab, [idx])
                return 0

            if st is not None:
                st.wait()
            lax.fori_loop(0, n_vec, body, 0, unroll=4)
            st = pltpu.async_copy(orow_v, out.at[s0 + i], st_sem)
        st.wait()

    return k


def kernel(price_data, date_idx, time_idx):
    batch = date_idx.shape[0]
    table = jnp.transpose(price_data, (1, 2, 0)).reshape(N_TIMES * F, N_DAYS)
    row0 = (time_idx - WINDOW) * F
    row0_arr = jnp.full((L,), row0, dtype=jnp.int32)
    didx = date_idx.astype(jnp.int32)
    out = _make_gather(batch)(table, didx, row0_arr)
    return jnp.transpose(out.reshape(WINDOW, F, batch), (2, 0, 1))


# speedup vs baseline: 61.2974x; 3.2844x over previous
import functools

import jax
import jax.numpy as jnp
from jax import lax
from jax.experimental import pallas as pl
from jax.experimental.pallas import tpu as pltpu
from jax.experimental.pallas import tpu_sc as plsc

N_DAYS = 16384
N_TIMES = 390
F = 8
WINDOW = 60
L = 16
NC, NS = 2, 16
NW = NC * NS
N_ROWS = WINDOW * F
ROWS_PER_W = N_ROWS // NW


def _make_gather(batch):
    n_vec = batch // L
    mesh = plsc.VectorSubcoreMesh(
        core_axis_name="c", subcore_axis_name="s",
        num_cores=NC, num_subcores=NS)

    @functools.partial(
        pl.kernel,
        mesh=mesh,
        out_type=jax.ShapeDtypeStruct((N_ROWS, batch), jnp.float32),
        scratch_types=[
            pltpu.VMEM((batch,), jnp.int32),
            pltpu.VMEM((batch,), jnp.float32),
            pltpu.VMEM((batch,), jnp.float32),
            pltpu.VMEM((batch,), jnp.float32),
            pltpu.VMEM((batch,), jnp.float32),
            pltpu.VMEM((L,), jnp.int32),
            pltpu.SemaphoreType.DMA,
            pltpu.SemaphoreType.DMA,
        ],
        compiler_params=pltpu.CompilerParams(needs_layout_passes=False),
    )
    def k(table, didx, row0, out, didx_v, slab_a, slab_b, orow_a, orow_b,
          r0_v, ld_sem, st_sem):
        wid = lax.axis_index("s") * NC + lax.axis_index("c")
        s0 = wid * ROWS_PER_W
        pltpu.sync_copy(didx.at[pl.ds(0, batch)], didx_v)
        pltpu.sync_copy(row0, r0_v)
        r0s = jnp.min(r0_v[...])
        slabs = [slab_a, slab_b]
        orows = [orow_a, orow_b]
        copies = [None, None]
        stores = [None, None]
        copies[0] = pltpu.async_copy(
            table.at[r0s + s0], slabs[0], ld_sem)
        for i in range(ROWS_PER_W):
            cur = i % 2
            copies[cur].wait()
            if i + 1 < ROWS_PER_W:
                copies[1 - cur] = pltpu.async_copy(
                    table.at[r0s + s0 + (i + 1)], slabs[1 - cur], ld_sem)
            slab = slabs[cur]
            orow_v = orows[cur]
            if stores[cur] is not None:
                stores[cur].wait()

            @plsc.parallel_loop(0, n_vec, 1, unroll=8)
            def body(v):
                idx = didx_v[pl.ds(v * L, L)]
                orow_v[pl.ds(v * L, L)] = plsc.load_gather(slab, [idx])

            stores[cur] = pltpu.async_copy(orow_v, out.at[s0 + i], st_sem)
        for s in stores:
            if s is not None:
                s.wait()

    return k


def kernel(price_data, date_idx, time_idx):
    batch = date_idx.shape[0]
    table = jnp.transpose(price_data, (1, 2, 0)).reshape(N_TIMES * F, N_DAYS)
    row0 = (time_idx - WINDOW) * F
    row0_arr = jnp.full((L,), row0, dtype=jnp.int32)
    didx = date_idx.astype(jnp.int32)
    out = _make_gather(batch)(table, didx, row0_arr)
    return jnp.transpose(out.reshape(WINDOW, F, batch), (2, 0, 1))
